# baseline (speedup 1.0000x reference)
import functools

import jax
import jax.numpy as jnp
from jax import lax
from jax.experimental import pallas as pl
from jax.experimental.pallas import tpu as pltpu
from jax.experimental.pallas import tpu_sc as plsc

_B, _F, _V, _D = 16384, 26, 100001, 64
_NC, _NS = 2, 16
_NW = _NC * _NS
_BPW = _B // _NW
_CHUNK = 128
_KPF = _BPW // _CHUNK
_NCHUNK = _F * _KPF
_NBUF = 8
_LAG = 4

_mesh = plsc.VectorSubcoreMesh(core_axis_name="c", subcore_axis_name="s")


@functools.partial(
    pl.kernel,
    mesh=_mesh,
    out_type=jax.ShapeDtypeStruct((_B, _F, _D), jnp.float32),
    scratch_types=[
        pltpu.VMEM((_F, _KPF, _CHUNK), jnp.int32),
        pltpu.VMEM((_NBUF, _CHUNK, _D), jnp.float32),
        pltpu.SemaphoreType.DMA((_NBUF,)),
        pltpu.SemaphoreType.DMA((_NBUF,)),
    ],
    compiler_params=pltpu.CompilerParams(use_tc_tiling_on_sc=False),
)
def _gather(xt_hbm, table3_hbm, out_hbm, xv, rows_v, gsem, ssem):
    wid = lax.axis_index("s") * _NC + lax.axis_index("c")
    b0 = wid * _BPW

    pltpu.sync_copy(xt_hbm.at[:, pl.ds(wid * _KPF, _KPF)], xv)

    def fire_gather(f, k, b):
        pltpu.async_copy(
            table3_hbm.at[f].at[xv.at[f, k]], rows_v.at[b], gsem.at[b])

    def wait_gather(f, k, b):
        pltpu.make_async_copy(
            table3_hbm.at[f].at[xv.at[f, k]], rows_v.at[b], gsem.at[b]).wait()

    def fire_store(f, k, b):
        pltpu.async_copy(
            rows_v.at[b], out_hbm.at[pl.ds(b0 + k * _CHUNK, _CHUNK), f],
            ssem.at[b])

    def wait_store(b):
        pltpu.make_async_copy(
            rows_v.at[b], out_hbm.at[pl.ds(b0, _CHUNK), 0], ssem.at[b]).wait()

    for t in range(_NCHUNK + _LAG):
        g = t
        c = t - _LAG
        if g < _NCHUNK:
            b = g % _NBUF
            if g >= _NBUF:
                wait_store(b)
            fire_gather(g // _KPF, g % _KPF, b)
        if c >= 0:
            b = c % _NBUF
            wait_gather(c // _KPF, c % _KPF, b)
            fire_store(c // _KPF, c % _KPF, b)

    for b in range(_NBUF):
        wait_store(b)


def kernel(x, tables):
    xt = x.T.reshape(_F, _B // _CHUNK, _CHUNK)
    out = _gather(xt, tables)
    return out, (x == 0)


# speedup vs baseline: 1.0760x; 1.0760x over previous
import functools

import jax
import jax.numpy as jnp
from jax import lax
from jax.experimental import pallas as pl
from jax.experimental.pallas import tpu as pltpu
from jax.experimental.pallas import tpu_sc as plsc

_B, _F, _V, _D = 16384, 26, 100001, 64
_NC, _NS = 2, 16
_NW = _NC * _NS
_BPW = _B // _NW
_XC = _B // 128

_mesh = plsc.VectorSubcoreMesh(core_axis_name="c", subcore_axis_name="s")


@functools.partial(
    pl.kernel,
    mesh=_mesh,
    out_type=jax.ShapeDtypeStruct((_B, _F, _D), jnp.float32),
    scratch_types=[
        pltpu.VMEM((_F, _BPW // 128, 128), jnp.int32),
        pltpu.SemaphoreType.DMA,
    ],
)
def _gather(xt_hbm, table3_hbm, out_hbm, xv, sem):
    wid = lax.axis_index("s") * _NC + lax.axis_index("c")
    b0 = wid * _BPW

    pltpu.sync_copy(xt_hbm.at[:, pl.ds(wid * (_BPW // 128), _BPW // 128)], xv)

    def group_body(t, carry):
        f = t >> 5
        g = t & 31
        row = xv.at[f, g >> 3]
        vec = row[pl.ds((g & 7) * 16, 16)]
        b = b0 + g * 16
        for l in range(16):
            v = vec[l]
            pltpu.async_copy(table3_hbm.at[f, v], out_hbm.at[b + l, f], sem)
        return carry

    lax.fori_loop(0, _F * (_BPW // 16), group_body, 0)

    def drain_body(j, carry):
        pltpu.make_async_copy(table3_hbm.at[0, 0], out_hbm.at[0, 0], sem).wait()
        return carry

    lax.fori_loop(0, _F * _BPW, drain_body, 0)


def kernel(x, tables):
    xt = x.T.reshape(_F, _XC, 128)
    out = _gather(xt, tables)
    return out, (x == 0)


# speedup vs baseline: 3.0176x; 2.8046x over previous
import functools

import jax
import jax.numpy as jnp
from jax import lax
from jax.experimental import pallas as pl
from jax.experimental.pallas import tpu as pltpu
from jax.experimental.pallas import tpu_sc as plsc

_B, _F, _V, _D = 16384, 26, 100001, 64
_NC, _NS = 2, 16
_NW = _NC * _NS
_BPW = _B // _NW
_CHUNK = 128
_KPF = _BPW // _CHUNK
_NCHUNK = _F * _KPF
_NBUF = 8
_LAG = 4

_mesh = plsc.VectorSubcoreMesh(core_axis_name="c", subcore_axis_name="s")


@functools.partial(
    pl.kernel,
    mesh=_mesh,
    out_type=jax.ShapeDtypeStruct((_B, _F, _D), jnp.float32),
    scratch_types=[
        pltpu.VMEM((_F, _KPF, _CHUNK), jnp.int32),
        pltpu.VMEM((_NBUF, _CHUNK, _D), jnp.float32),
        pltpu.SemaphoreType.DMA((_NBUF,)),
        pltpu.SemaphoreType.DMA((_NBUF,)),
    ],
    compiler_params=pltpu.CompilerParams(use_tc_tiling_on_sc=False),
)
def _gather(xt_hbm, *rest):
    tables = rest[:_F]
    out_hbm, xv, rows_v, gsem, ssem = rest[_F:]
    wid = lax.axis_index("s") * _NC + lax.axis_index("c")
    b0 = wid * _BPW

    pltpu.sync_copy(xt_hbm.at[:, pl.ds(wid * _KPF, _KPF)], xv)

    def fire_gather(f, k, b):
        pltpu.async_copy(
            tables[f].at[xv.at[f, k]], rows_v.at[b], gsem.at[b])

    def wait_gather(f, k, b):
        pltpu.make_async_copy(
            tables[f].at[xv.at[f, k]], rows_v.at[b], gsem.at[b]).wait()

    def fire_store(f, k, b):
        pltpu.async_copy(
            rows_v.at[b], out_hbm.at[pl.ds(b0 + k * _CHUNK, _CHUNK), f],
            ssem.at[b])

    def wait_store(b):
        pltpu.make_async_copy(
            rows_v.at[b], out_hbm.at[pl.ds(b0, _CHUNK), 0], ssem.at[b]).wait()

    for t in range(_NCHUNK + _LAG):
        g = t
        c = t - _LAG
        if g < _NCHUNK:
            b = g % _NBUF
            if g >= _NBUF:
                wait_store(b)
            fire_gather(g // _KPF, g % _KPF, b)
        if c >= 0:
            b = c % _NBUF
            wait_gather(c // _KPF, c % _KPF, b)
            fire_store(c // _KPF, c % _KPF, b)

    for b in range(_NBUF):
        wait_store(b)


def kernel(x, tables):
    xt = x.T.reshape(_F, _B // _CHUNK, _CHUNK)
    out = _gather(xt, *[tables[f] for f in range(_F)])
    return out, (x == 0)
